# baseline (speedup 1.0000x reference)
import functools

import jax
import jax.numpy as jnp
from jax import lax
from jax.experimental import pallas as pl
from jax.experimental.pallas import tpu as pltpu
from jax.experimental.pallas import tpu_sc as plsc

N_FIELDS = 8
VOCAB_SZ = 1000000
BATCH_SZ = 16384
N_NLP = 5
LAMBDA_LLM = 0.1

NC = 2
NS = 16
LANES = 16
NW = NC * NS

ROWS_W = BATCH_SZ // NW
GATHERS_W = ROWS_W * N_FIELDS
BBLK = 128
NBLK_W = ROWS_W // BBLK
LLM_W = ROWS_W * N_NLP

_mesh = plsc.VectorSubcoreMesh(core_axis_name="c", subcore_axis_name="s")


@functools.partial(
    pl.kernel,
    mesh=_mesh,
    compiler_params=pltpu.CompilerParams(needs_layout_passes=False),
    out_type=jax.ShapeDtypeStruct((BATCH_SZ,), jnp.float32),
    scratch_types=[
        pltpu.VMEM((GATHERS_W,), jnp.int32),
        pltpu.VMEM((GATHERS_W,), jnp.float32),
        pltpu.VMEM((LLM_W,), jnp.float32),
        pltpu.VMEM((LANES,), jnp.float32),
        pltpu.VMEM((ROWS_W,), jnp.float32),
        pltpu.SemaphoreType.DMA,
        pltpu.SemaphoreType.DMA,
        pltpu.SemaphoreType.DMA,
        pltpu.SemaphoreType.DMA,
    ],
)
def _lr_lookup(x_hbm, llm_hbm, tab_hbm, bias_hbm,
               out_hbm, x_v, vals_v, llm_v, bias_v, out_v,
               sem0, sem1, sem2, sem3):
    sems = (sem0, sem1, sem2, sem3)
    tabs = [tab_hbm.at[f, 0] for f in range(N_FIELDS)]
    wid = lax.axis_index("s") * NC + lax.axis_index("c")
    base = wid * ROWS_W

    pltpu.sync_copy(x_hbm.at[pl.ds(wid * GATHERS_W, GATHERS_W)], x_v)

    for jb in range(NBLK_W):
        for f in range(N_FIELDS):
            pltpu.async_copy(
                tabs[f].at[x_v.at[pl.ds(jb * BBLK * N_FIELDS + f * BBLK, BBLK)]],
                vals_v.at[pl.ds(f * ROWS_W + jb * BBLK, BBLK)],
                sems[jb],
            )

    for j in range(N_NLP):
        pltpu.sync_copy(
            llm_hbm.at[pl.ds(j * BATCH_SZ + base, ROWS_W)],
            llm_v.at[pl.ds(j * ROWS_W, ROWS_W)],
        )
    pltpu.sync_copy(bias_hbm, bias_v)

    bias_vec = bias_v[...]

    def accum(i, carry):
        acc = bias_vec
        for f in range(N_FIELDS):
            acc = acc + vals_v[pl.ds(f * ROWS_W + i * LANES, LANES)]
        ls = llm_v[pl.ds(i * LANES, LANES)]
        for j in range(1, N_NLP):
            ls = ls + llm_v[pl.ds(j * ROWS_W + i * LANES, LANES)]
        out_v[pl.ds(i * LANES, LANES)] = acc + ls * LAMBDA_LLM
        return carry

    for jb in range(NBLK_W):
        pltpu.make_async_copy(
            tab_hbm.at[0, 0, pl.ds(0, BBLK * N_FIELDS)],
            vals_v.at[pl.ds(0, BBLK * N_FIELDS)],
            sems[jb],
        ).wait()
        lax.fori_loop(jb * BBLK // LANES, (jb + 1) * BBLK // LANES, accum, 0)

    pltpu.sync_copy(out_v, out_hbm.at[pl.ds(base, ROWS_W)])


def kernel(X, llm_emb_1, tables, bias):
    x_lin = X.reshape(BATCH_SZ // BBLK, BBLK, N_FIELDS).transpose(0, 2, 1).reshape(-1)
    llm_lin = llm_emb_1.transpose(1, 2, 0).reshape(-1)
    t3 = tables.transpose(0, 2, 1)
    bias16 = jnp.broadcast_to(bias, (LANES,))
    out = _lr_lookup(x_lin, llm_lin, t3, bias16)
    return out.reshape(BATCH_SZ, 1)


# speedup vs baseline: 1.0733x; 1.0733x over previous
import functools

import jax
import jax.numpy as jnp
from jax import lax
from jax.experimental import pallas as pl
from jax.experimental.pallas import tpu as pltpu
from jax.experimental.pallas import tpu_sc as plsc

N_FIELDS = 8
VOCAB_SZ = 1000000
BATCH_SZ = 16384
N_NLP = 5
LAMBDA_LLM = 0.1

NC = 2
NS = 16
LANES = 16
NW = NC * NS

ROWS_W = BATCH_SZ // NW
GATHERS_W = ROWS_W * N_FIELDS
BBLK = 128
NBLK_W = ROWS_W // BBLK
LLM_W = ROWS_W * N_NLP

_mesh = plsc.VectorSubcoreMesh(core_axis_name="c", subcore_axis_name="s")


@functools.partial(
    pl.kernel,
    mesh=_mesh,
    compiler_params=pltpu.CompilerParams(needs_layout_passes=False),
    out_type=jax.ShapeDtypeStruct((BATCH_SZ,), jnp.float32),
    scratch_types=[
        pltpu.VMEM((GATHERS_W,), jnp.int32),
        pltpu.VMEM((GATHERS_W,), jnp.float32),
        pltpu.VMEM((LLM_W,), jnp.float32),
        pltpu.VMEM((LANES,), jnp.float32),
        pltpu.VMEM((ROWS_W,), jnp.float32),
        pltpu.SemaphoreType.DMA,
        pltpu.SemaphoreType.DMA,
        pltpu.SemaphoreType.DMA,
        pltpu.SemaphoreType.DMA,
    ],
)
def _lr_lookup(x_hbm, llm_hbm, tab_hbm, bias_hbm,
               out_hbm, x_v, vals_v, llm_v, bias_v, out_v,
               sem0, sem1, sem2, sem3):
    sems = (sem0, sem1, sem2, sem3)
    tabs = [tab_hbm.at[f, 0] for f in range(N_FIELDS)]
    wid = lax.axis_index("s") * NC + lax.axis_index("c")
    base = wid * ROWS_W

    pltpu.sync_copy(x_hbm.at[pl.ds(wid * GATHERS_W, GATHERS_W)], x_v)

    for jb in range(NBLK_W):
        for f in range(N_FIELDS):
            pltpu.async_copy(
                tabs[f].at[x_v.at[pl.ds(jb * BBLK * N_FIELDS + f * BBLK, BBLK)]],
                vals_v.at[pl.ds(f * ROWS_W + jb * BBLK, BBLK)],
                sem0,
            )

    for j in range(N_NLP):
        pltpu.sync_copy(
            llm_hbm.at[pl.ds(j * BATCH_SZ + base, ROWS_W)],
            llm_v.at[pl.ds(j * ROWS_W, ROWS_W)],
        )
    pltpu.sync_copy(bias_hbm, bias_v)

    bias_vec = bias_v[...]

    def accum(i, carry):
        acc = bias_vec
        for f in range(N_FIELDS):
            acc = acc + vals_v[pl.ds(f * ROWS_W + i * LANES, LANES)]
        ls = llm_v[pl.ds(i * LANES, LANES)]
        for j in range(1, N_NLP):
            ls = ls + llm_v[pl.ds(j * ROWS_W + i * LANES, LANES)]
        out_v[pl.ds(i * LANES, LANES)] = acc + ls * LAMBDA_LLM
        return carry

    pltpu.make_async_copy(tab_hbm.at[0, 0, pl.ds(0, GATHERS_W)], vals_v, sem0).wait()
    lax.fori_loop(0, ROWS_W // LANES, accum, 0)

    pltpu.sync_copy(out_v, out_hbm.at[pl.ds(base, ROWS_W)])


def kernel(X, llm_emb_1, tables, bias):
    x_lin = X.reshape(BATCH_SZ // BBLK, BBLK, N_FIELDS).transpose(0, 2, 1).reshape(-1)
    llm_lin = llm_emb_1.transpose(1, 2, 0).reshape(-1)
    t3 = tables.transpose(0, 2, 1)
    bias16 = jnp.broadcast_to(bias, (LANES,))
    out = _lr_lookup(x_lin, llm_lin, t3, bias16)
    return out.reshape(BATCH_SZ, 1)


# speedup vs baseline: 1.0768x; 1.0033x over previous
import functools

import jax
import jax.numpy as jnp
from jax import lax
from jax.experimental import pallas as pl
from jax.experimental.pallas import tpu as pltpu
from jax.experimental.pallas import tpu_sc as plsc

N_FIELDS = 8
VOCAB_SZ = 1000000
BATCH_SZ = 16384
N_NLP = 5
LAMBDA_LLM = 0.1

NC = 2
NS = 16
LANES = 16
NW = NC * NS

ROWS_W = BATCH_SZ // NW
GATHERS_W = ROWS_W * N_FIELDS
BBLK = 128
NBLK_W = ROWS_W // BBLK
LLM_W = ROWS_W * N_NLP

_mesh = plsc.VectorSubcoreMesh(core_axis_name="c", subcore_axis_name="s")


@functools.partial(
    pl.kernel,
    mesh=_mesh,
    compiler_params=pltpu.CompilerParams(needs_layout_passes=False),
    out_type=jax.ShapeDtypeStruct((BATCH_SZ,), jnp.float32),
    scratch_types=[
        pltpu.VMEM((GATHERS_W,), jnp.int32),
        pltpu.VMEM((GATHERS_W,), jnp.float32),
        pltpu.VMEM((LLM_W,), jnp.float32),
        pltpu.VMEM((LANES,), jnp.float32),
        pltpu.VMEM((ROWS_W,), jnp.float32),
        pltpu.SemaphoreType.DMA,
    ],
)
def _lr_lookup(x_hbm, llm_hbm, tab_hbm, bias_hbm,
               out_hbm, x_v, vals_v, llm_v, bias_v, out_v, sem):
    tabs = [tab_hbm.at[f, 0] for f in range(N_FIELDS)]
    wid = lax.axis_index("s") * NC + lax.axis_index("c")
    base = wid * ROWS_W

    pltpu.sync_copy(x_hbm.at[pl.ds(wid * GATHERS_W, GATHERS_W)], x_v)

    for jb in range(NBLK_W):
        for f in range(N_FIELDS):
            pltpu.async_copy(
                tabs[f].at[x_v.at[pl.ds(jb * BBLK * N_FIELDS + f * BBLK, BBLK)]],
                vals_v.at[pl.ds(f * ROWS_W + jb * BBLK, BBLK)],
                sem,
            )

    for j in range(N_NLP):
        pltpu.sync_copy(
            llm_hbm.at[pl.ds(j * BATCH_SZ + base, ROWS_W)],
            llm_v.at[pl.ds(j * ROWS_W, ROWS_W)],
        )
    pltpu.sync_copy(bias_hbm, bias_v)

    bias_vec = bias_v[...]

    pltpu.make_async_copy(tab_hbm.at[0, 0, pl.ds(0, GATHERS_W)], vals_v, sem).wait()

    @plsc.parallel_loop(0, ROWS_W, step=LANES, unroll=4)
    def accum(r):
        acc = bias_vec
        for f in range(N_FIELDS):
            acc = acc + vals_v[pl.ds(f * ROWS_W + r, LANES)]
        ls = llm_v[pl.ds(r, LANES)]
        for j in range(1, N_NLP):
            ls = ls + llm_v[pl.ds(j * ROWS_W + r, LANES)]
        out_v[pl.ds(r, LANES)] = acc + ls * LAMBDA_LLM

    pltpu.sync_copy(out_v, out_hbm.at[pl.ds(base, ROWS_W)])


def kernel(X, llm_emb_1, tables, bias):
    x_lin = X.reshape(BATCH_SZ // BBLK, BBLK, N_FIELDS).transpose(0, 2, 1).reshape(-1)
    llm_lin = llm_emb_1.transpose(1, 2, 0).reshape(-1)
    t3 = tables.transpose(0, 2, 1)
    bias16 = jnp.broadcast_to(bias, (LANES,))
    out = _lr_lookup(x_lin, llm_lin, t3, bias16)
    return out.reshape(BATCH_SZ, 1)


# speedup vs baseline: 1.0793x; 1.0023x over previous
import functools

import jax
import jax.numpy as jnp
from jax import lax
from jax.experimental import pallas as pl
from jax.experimental.pallas import tpu as pltpu
from jax.experimental.pallas import tpu_sc as plsc

N_FIELDS = 8
VOCAB_SZ = 1000000
BATCH_SZ = 16384
N_NLP = 5
LAMBDA_LLM = 0.1

NC = 2
NS = 16
LANES = 16
NW = NC * NS

ROWS_W = BATCH_SZ // NW
GATHERS_W = ROWS_W * N_FIELDS
BBLK = 128
NBLK_W = ROWS_W // BBLK
LLM_W = ROWS_W * N_NLP

_mesh = plsc.VectorSubcoreMesh(core_axis_name="c", subcore_axis_name="s")


@functools.partial(
    pl.kernel,
    mesh=_mesh,
    compiler_params=pltpu.CompilerParams(needs_layout_passes=False),
    out_type=jax.ShapeDtypeStruct((BATCH_SZ,), jnp.float32),
    scratch_types=[
        pltpu.VMEM((GATHERS_W,), jnp.int32),
        pltpu.VMEM((GATHERS_W,), jnp.float32),
        pltpu.VMEM((LLM_W,), jnp.float32),
        pltpu.VMEM((LANES,), jnp.float32),
        pltpu.VMEM((ROWS_W,), jnp.float32),
        pltpu.SemaphoreType.DMA,
        pltpu.SemaphoreType.DMA,
        pltpu.SemaphoreType.DMA,
        pltpu.SemaphoreType.DMA,
        pltpu.SemaphoreType.DMA,
    ],
)
def _lr_lookup(x_hbm, llm_hbm, tab_hbm, bias_hbm,
               out_hbm, x_v, vals_v, llm_v, bias_v, out_v,
               sem, sx0, sx1, sx2, sx3):
    sem_x = (sx0, sx1, sx2, sx3)
    tabs = [tab_hbm.at[f, 0] for f in range(N_FIELDS)]
    wid = lax.axis_index("s") * NC + lax.axis_index("c")
    base = wid * ROWS_W

    XBLK = BBLK * N_FIELDS
    for jb in range(NBLK_W):
        pltpu.async_copy(
            x_hbm.at[pl.ds(wid * GATHERS_W + jb * XBLK, XBLK)],
            x_v.at[pl.ds(jb * XBLK, XBLK)],
            sem_x[jb],
        )
    for jb in range(NBLK_W):
        pltpu.make_async_copy(
            x_hbm.at[pl.ds(0, XBLK)], x_v.at[pl.ds(0, XBLK)], sem_x[jb],
        ).wait()
        for f in range(N_FIELDS):
            pltpu.async_copy(
                tabs[f].at[x_v.at[pl.ds(jb * BBLK * N_FIELDS + f * BBLK, BBLK)]],
                vals_v.at[pl.ds(f * ROWS_W + jb * BBLK, BBLK)],
                sem,
            )

    for j in range(N_NLP):
        pltpu.sync_copy(
            llm_hbm.at[pl.ds(j * BATCH_SZ + base, ROWS_W)],
            llm_v.at[pl.ds(j * ROWS_W, ROWS_W)],
        )
    pltpu.sync_copy(bias_hbm, bias_v)

    bias_vec = bias_v[...]

    pltpu.make_async_copy(tab_hbm.at[0, 0, pl.ds(0, GATHERS_W)], vals_v, sem).wait()

    @plsc.parallel_loop(0, ROWS_W, step=LANES, unroll=4)
    def accum(r):
        acc = bias_vec
        for f in range(N_FIELDS):
            acc = acc + vals_v[pl.ds(f * ROWS_W + r, LANES)]
        ls = llm_v[pl.ds(r, LANES)]
        for j in range(1, N_NLP):
            ls = ls + llm_v[pl.ds(j * ROWS_W + r, LANES)]
        out_v[pl.ds(r, LANES)] = acc + ls * LAMBDA_LLM

    pltpu.sync_copy(out_v, out_hbm.at[pl.ds(base, ROWS_W)])


def kernel(X, llm_emb_1, tables, bias):
    x_lin = X.reshape(BATCH_SZ // BBLK, BBLK, N_FIELDS).transpose(0, 2, 1).reshape(-1)
    llm_lin = llm_emb_1.transpose(1, 2, 0).reshape(-1)
    t3 = tables.transpose(0, 2, 1)
    bias16 = jnp.broadcast_to(bias, (LANES,))
    out = _lr_lookup(x_lin, llm_lin, t3, bias16)
    return out.reshape(BATCH_SZ, 1)
